# Initial kernel scaffold; baseline (speedup 1.0000x reference)
import jax
import jax.numpy as jnp
from jax.experimental import pallas as pl


def kernel(x, tgt):
    raise NotImplementedError("write your pallas kernel here")


# speedup vs baseline: 6.1671x; 6.1671x over previous
import math

import jax
import jax.numpy as jnp
from jax.experimental import pallas as pl
from jax.experimental.pallas import tpu as pltpu

_SIZE = 32000
_N = 4096
_SMOOTHING = 0.1
_EPS = _SMOOTHING / (_SIZE - 2)
_CONF = 1.0 - _SMOOTHING
_C0 = _N * (_SIZE - 1) * _EPS * math.log(_EPS)
_DCONST = _CONF * math.log(_CONF) - _EPS * math.log(_EPS)

_BR = 512
_BC = 3200
_RI = _N // _BR
_CJ = _SIZE // _BC


def _body(tgt_ref, x_ref, out_ref, acc_ref):
    i = pl.program_id(0)
    j = pl.program_id(1)

    @pl.when((i == 0) & (j == 0))
    def _init():
        acc_ref[0] = 0.0
        acc_ref[1] = 0.0
        acc_ref[2] = 0.0
        acc_ref[3] = 0.0

    xt = x_ref[...]
    acc_ref[0] += jnp.sum(xt)

    tgt_col = tgt_ref[0, 0, :].reshape(_BR, 1)
    col = jax.lax.broadcasted_iota(jnp.int32, (_BR, _BC), 1) + j * _BC
    mask = (col == tgt_col) & (tgt_col != 0)
    acc_ref[1] += jnp.sum(jnp.where(mask, xt, 0.0))

    @pl.when(j == 0)
    def _first_col_block():
        acc_ref[2] += jnp.sum(xt[:, 0:1])
        acc_ref[3] += jnp.sum((tgt_ref[0, 0, :] != 0).astype(jnp.float32))

    @pl.when((i == _RI - 1) & (j == _CJ - 1))
    def _finalize():
        out_ref[0, 0] = (_C0
                         - _EPS * acc_ref[0]
                         + _EPS * acc_ref[2]
                         + _DCONST * acc_ref[3]
                         - (_CONF - _EPS) * acc_ref[1])


def kernel(x, tgt):
    tgt3 = tgt.astype(jnp.int32).reshape(_RI, 1, _BR)
    out = pl.pallas_call(
        _body,
        grid=(_RI, _CJ),
        in_specs=[
            pl.BlockSpec((1, 1, _BR), lambda i, j: (i, 0, 0)),
            pl.BlockSpec((_BR, _BC), lambda i, j: (i, j)),
        ],
        out_specs=pl.BlockSpec(memory_space=pltpu.SMEM),
        out_shape=jax.ShapeDtypeStruct((1, 1), jnp.float32),
        scratch_shapes=[pltpu.SMEM((4,), jnp.float32)],
    )(tgt3, x)
    return out[0, 0]
